# baseline (speedup 1.0000x reference)
import functools

import jax
import jax.numpy as jnp
from jax import lax
from jax.experimental import pallas as pl
from jax.experimental.pallas import tpu as pltpu
from jax.experimental.pallas import tpu_sc as plsc

_NW = 32
_CH = 128
_ALIGN = _NW * _CH

def _unpack_bf16_pairs(u):
    lo = lax.bitcast_convert_type(u << 16, jnp.float32)
    hi = lax.bitcast_convert_type(u & jnp.int32(-65536), jnp.float32)
    return jnp.concatenate([lo, hi], axis=1)


def _proj_body(src_ref, wt_ref, wa_ref, wb_ref, bt_ref, ba_ref, bb_ref,
               pt_ref, pa_ref, pb_ref):
    s = src_ref[...]
    pt_ref[...] = jnp.dot(s, wt_ref[...], preferred_element_type=jnp.float32) + bt_ref[...]
    pa_ref[...] = jnp.dot(s, wa_ref[...], preferred_element_type=jnp.float32) + ba_ref[...]
    pb_ref[...] = jnp.dot(s, wb_ref[...], preferred_element_type=jnp.float32) + bb_ref[...]


def _project_tables(src, wct, wca, wcb, bt, ba, bb):
    B, H = src.shape[0], wct.shape[1]
    out = jax.ShapeDtypeStruct((B, H), jnp.float32)
    return pl.pallas_call(
        _proj_body,
        out_shape=[out, out, out],
    )(src, wct, wca, wcb, bt, ba, bb)


def _make_gather(n_pad, H):
    total_chunks = n_pad // _CH
    chunks_pw = total_chunks // _NW
    mesh = plsc.VectorSubcoreMesh(core_axis_name="c", subcore_axis_name="s")

    @functools.partial(
        pl.kernel,
        out_type=jax.ShapeDtypeStruct((n_pad, H // 2), jnp.int32),
        mesh=mesh,
        scratch_types=[
            pltpu.VMEM((chunks_pw * _CH,), jnp.int32),
            pltpu.VMEM((_CH, H), jnp.float32),
            pltpu.VMEM((_CH, H), jnp.float32),
            pltpu.VMEM((_CH, H // 2), jnp.int32),
            pltpu.VMEM((_CH, H // 2), jnp.int32),
            pltpu.VMEM_SHARED((4096, H), jnp.float32),
            pltpu.SemaphoreType.DMA,
            pltpu.SemaphoreType.DMA,
            pltpu.SemaphoreType.DMA,
            pltpu.SemaphoreType.DMA,
        ],
    )
    def gath(table_hbm, idx_hbm, out_hbm, idx_v, fbuf0, fbuf1, pbuf0, pbuf1,
             table_sh, sem_g0, sem_g1, sem_w0, sem_w1):
        sid = lax.axis_index("s")
        wid = sid * 2 + lax.axis_index("c")
        rows_pw = chunks_pw * _CH
        base = wid * chunks_pw

        @pl.when(sid == 0)
        def _():
            pltpu.sync_copy(table_hbm, table_sh)

        pltpu.sync_copy(idx_hbm.at[pl.ds(wid * rows_pw, rows_pw)], idx_v)
        plsc.subcore_barrier()

        def start_gather(c, buf, sem):
            pltpu.async_copy(table_sh.at[idx_v.at[pl.ds(c * _CH, _CH)]], buf, sem)

        def wait_gather(c, buf, sem):
            pltpu.make_async_copy(
                table_sh.at[idx_v.at[pl.ds(c * _CH, _CH)]], buf, sem).wait()

        def start_write(c, buf, sem):
            pltpu.async_copy(buf, out_hbm.at[pl.ds((base + c) * _CH, _CH), :], sem)

        def wait_write(c, buf, sem):
            pltpu.make_async_copy(
                buf, out_hbm.at[pl.ds((base + c) * _CH, _CH), :], sem).wait()

        def pack_chunk(fbuf, pbuf):
            def rowbody(r4, carry):
                for u in range(4):
                    r = r4 * 4 + u
                    for jg in range(H // 32):
                        a = fbuf[r, pl.ds(16 * jg, 16)]
                        b = fbuf[r, pl.ds(H // 2 + 16 * jg, 16)]
                        ab = lax.bitcast_convert_type(a, jnp.int32)
                        bb = lax.bitcast_convert_type(b, jnp.int32)
                        ra = ab + 0x7FFF + ((ab >> 16) & 1)
                        rb = bb + 0x7FFF + ((bb >> 16) & 1)
                        pbuf[r, pl.ds(16 * jg, 16)] = (
                            ((ra >> 16) & 0xFFFF) | (rb & jnp.int32(-65536)))
                return carry

            lax.fori_loop(0, _CH // 4, rowbody, 0)

        n_pairs = chunks_pw // 2
        tail = chunks_pw % 2
        start_gather(0, fbuf0, sem_g0)

        def body(g, carry):
            c0 = 2 * g
            wait_gather(c0, fbuf0, sem_g0)
            start_gather(c0 + 1, fbuf1, sem_g1)
            pack_chunk(fbuf0, pbuf0)

            @pl.when(g >= 1)
            def _():
                wait_write(c0 - 1, pbuf1, sem_w1)

            start_write(c0, pbuf0, sem_w0)
            wait_gather(c0 + 1, fbuf1, sem_g1)
            cn = jnp.minimum(c0 + 2, chunks_pw - 1)
            start_gather(cn, fbuf0, sem_g0)
            pack_chunk(fbuf1, pbuf1)
            wait_write(c0, pbuf0, sem_w0)
            start_write(c0 + 1, pbuf1, sem_w1)
            return carry

        if n_pairs > 0:
            lax.fori_loop(0, n_pairs, body, 0)
        last = chunks_pw - 1
        wait_gather(last, fbuf0, sem_g0)
        if tail:
            pack_chunk(fbuf0, pbuf0)
            start_write(last, pbuf0, sem_w0)
            wait_write(last, pbuf0, sem_w0)
        if n_pairs > 0:
            wait_write(last - tail, pbuf1, sem_w1)

    return gath


def _gather_ctx(table, idx):
    n = idx.shape[0]
    n_pad = -(-n // _ALIGN) * _ALIGN
    idx_pad = jnp.concatenate(
        [idx.astype(jnp.int32), jnp.zeros((n_pad - n,), jnp.int32)]
    )
    return _make_gather(n_pad, table.shape[1])(table, idx_pad)


def _accum(out_ref, i, loss_s, acc_s):
    l2 = lax.broadcasted_iota(jnp.int32, (1, 128), 1)
    upd = jnp.where(l2 == 0, loss_s, 0.0) + jnp.where(l2 == 1, acc_s, 0.0)

    @pl.when(i == 0)
    def _():
        out_ref[...] = jnp.zeros_like(out_ref)

    out_ref[...] += upd


def _softmax_body(T, N, vec_ref, ctx_ref, lab_ref, w1_ref, w2_ref, b2_ref, out_ref):
    i = pl.program_id(0)
    h = jnp.dot(vec_ref[...], w1_ref[...], preferred_element_type=jnp.float32)
    h = jnp.maximum(h + _unpack_bf16_pairs(ctx_ref[...]), 0.0)
    lt = lax.dot_general(w2_ref[...], h, (((1,), (1,)), ((), ())),
                         preferred_element_type=jnp.float32) + b2_ref[...]
    C = lt.shape[0]
    lab = lab_ref[...]
    sub = lax.broadcasted_iota(jnp.int32, lt.shape, 0)
    lse = jnp.log(jnp.sum(jnp.exp(lt), axis=0, keepdims=True))
    gold = jnp.sum(jnp.where(sub == lab, lt, 0.0), axis=0, keepdims=True)
    m = jnp.max(lt, axis=0, keepdims=True)
    amax = jnp.min(jnp.where(lt == m, sub, C), axis=0, keepdims=True)
    col = lax.broadcasted_iota(jnp.int32, (1, T), 1) + i * T
    valid = col < N
    loss_s = jnp.sum(jnp.where(valid, lse - gold, 0.0))
    acc_s = jnp.sum(jnp.where(valid & (amax == lab), 1.0, 0.0))
    _accum(out_ref, i, loss_s, acc_s)


def _topo_body(T, N, vec_ref, ctx_ref, lab_ref, w1_ref, w2_ref, b2_ref, out_ref):
    i = pl.program_id(0)
    h = jnp.dot(vec_ref[...], w1_ref[...], preferred_element_type=jnp.float32)
    h = jnp.maximum(h + _unpack_bf16_pairs(ctx_ref[...]), 0.0)
    z = lax.dot_general(w2_ref[...], h, (((1,), (1,)), ((), ())),
                        preferred_element_type=jnp.float32) + b2_ref[...]
    y = lab_ref[...]
    loss_row = jnp.maximum(z, 0.0) - z * y + jnp.log1p(jnp.exp(-jnp.abs(z)))
    corr = (z >= 0.0) == (y >= 0.5)
    col = lax.broadcasted_iota(jnp.int32, (1, T), 1) + i * T
    valid = col < N
    loss_s = jnp.sum(jnp.where(valid, loss_row, 0.0))
    acc_s = jnp.sum(jnp.where(valid & corr, 1.0, 0.0))
    _accum(out_ref, i, loss_s, acc_s)


def _fused_stream(body, vecs, ctx, lab2d, w1t, w2, b2, T=4096):
    N, D = vecs.shape
    H = w1t.shape[1]
    grid = -(-N // T)
    return pl.pallas_call(
        functools.partial(body, T, N),
        grid=(grid,),
        in_specs=[
            pl.BlockSpec((T, D), lambda i: (i, 0)),
            pl.BlockSpec((T, ctx.shape[1]), lambda i: (i, 0)),
            pl.BlockSpec((1, T), lambda i: (0, i)),
            pl.BlockSpec(w1t.shape, lambda i: (0, 0)),
            pl.BlockSpec(w2.shape, lambda i: (0, 0)),
            pl.BlockSpec(b2.shape, lambda i: (0, 0)),
        ],
        out_specs=pl.BlockSpec((1, 128), lambda i: (0, 0)),
        out_shape=jax.ShapeDtypeStruct((1, 128), jnp.float32),
    )(vecs, ctx, lab2d, w1t, w2, b2)


def kernel(src_graph_vecs, topo_vecs, atom_vecs, bond_vecs,
           W_t1, b_t1, W_t2, b_t2, W_a1, b_a1, W_a2, b_a2,
           W_b1, b_b1, W_b2, b_b2,
           topo_idx, atom_idx, bond_idx,
           topo_labels, atom_labels, bond_labels):
    B, L = src_graph_vecs.shape
    H = W_t1.shape[0]
    Dt = topo_vecs.shape[1]
    Da = atom_vecs.shape[1]
    Db = bond_vecs.shape[1]
    V = W_a2.shape[0]
    NBOND = W_b2.shape[0]
    NT, NA, NB = topo_idx.shape[0], atom_idx.shape[0], bond_idx.shape[0]

    pt, pa, pb = _project_tables(
        src_graph_vecs,
        W_t1[:, Dt:].T, W_a1[:, Da:].T, W_b1[:, Db:].T,
        b_t1[None, :], b_a1[None, :], b_b1[None, :],
    )

    ctx_a = _gather_ctx(pa, atom_idx)
    ctx_t = _gather_ctx(pt, topo_idx)
    ctx_b = _gather_ctx(pb, bond_idx)

    w2b = jnp.zeros((8, H), jnp.float32).at[:NBOND, :].set(W_b2)
    b2b = jnp.full((8, 1), -1e30, jnp.float32).at[:NBOND, 0].set(b_b2)

    a_out = _fused_stream(_softmax_body, atom_vecs, ctx_a,
                          atom_labels.astype(jnp.int32)[None, :],
                          W_a1[:, :Da].T, W_a2, b_a2[:, None])
    t_out = _fused_stream(_topo_body, topo_vecs, ctx_t,
                          topo_labels.astype(jnp.float32)[None, :],
                          W_t1[:, :Dt].T, W_t2, b_t2[:, None])
    b_out = _fused_stream(_softmax_body, bond_vecs, ctx_b,
                          bond_labels.astype(jnp.int32)[None, :],
                          W_b1[:, :Db].T, w2b, b2b)

    loss = (t_out[0, 0] + a_out[0, 0] + b_out[0, 0]) / B
    topo_acc = t_out[0, 1] / NT
    atom_acc = a_out[0, 1] / NA
    bond_acc = b_out[0, 1] / NB
    return (loss, atom_acc, topo_acc, bond_acc)


# speedup vs baseline: 1.0875x; 1.0875x over previous
import functools

import jax
import jax.numpy as jnp
from jax import lax
from jax.experimental import pallas as pl
from jax.experimental.pallas import tpu as pltpu
from jax.experimental.pallas import tpu_sc as plsc

_NW = 32
_CH = 128
_ALIGN = _NW * _CH


def _proj_body(src_ref, wt_ref, wa_ref, wb_ref, bt_ref, ba_ref, bb_ref,
               pt_ref, pa_ref, pb_ref):
    s = src_ref[...]
    pt_ref[...] = jnp.dot(s, wt_ref[...], preferred_element_type=jnp.float32) + bt_ref[...]
    pa_ref[...] = jnp.dot(s, wa_ref[...], preferred_element_type=jnp.float32) + ba_ref[...]
    pb_ref[...] = jnp.dot(s, wb_ref[...], preferred_element_type=jnp.float32) + bb_ref[...]


def _project_tables(src, wct, wca, wcb, bt, ba, bb):
    B, H = src.shape[0], wct.shape[1]
    out = jax.ShapeDtypeStruct((B, H), jnp.float32)
    return pl.pallas_call(
        _proj_body,
        out_shape=[out, out, out],
    )(src, wct, wca, wcb, bt, ba, bb)


def _make_gather(n_pad, H):
    total_chunks = n_pad // _CH
    chunks_pw = total_chunks // _NW
    mesh = plsc.VectorSubcoreMesh(core_axis_name="c", subcore_axis_name="s")

    @functools.partial(
        pl.kernel,
        out_type=jax.ShapeDtypeStruct((n_pad, H), jnp.float32),
        mesh=mesh,
        scratch_types=[
            pltpu.VMEM((chunks_pw * _CH,), jnp.int32),
            pltpu.VMEM((_CH, H), jnp.float32),
            pltpu.VMEM((_CH, H), jnp.float32),
            pltpu.VMEM_SHARED((4096, H), jnp.float32),
            pltpu.SemaphoreType.DMA,
            pltpu.SemaphoreType.DMA,
            pltpu.SemaphoreType.DMA,
            pltpu.SemaphoreType.DMA,
        ],
    )
    def gath(table_hbm, idx_hbm, out_hbm, idx_v, buf0, buf1, table_sh,
             sem_g0, sem_g1, sem_w0, sem_w1):
        sid = lax.axis_index("s")
        wid = sid * 2 + lax.axis_index("c")
        rows_pw = chunks_pw * _CH
        base = wid * chunks_pw

        @pl.when(sid == 0)
        def _():
            pltpu.sync_copy(table_hbm, table_sh)

        pltpu.sync_copy(idx_hbm.at[pl.ds(wid * rows_pw, rows_pw)], idx_v)
        plsc.subcore_barrier()

        def start_gather(c, buf, sem):
            pltpu.async_copy(table_sh.at[idx_v.at[pl.ds(c * _CH, _CH)]], buf, sem)

        def wait_gather(c, buf, sem):
            pltpu.make_async_copy(
                table_sh.at[idx_v.at[pl.ds(c * _CH, _CH)]], buf, sem).wait()

        def start_write(c, buf, sem):
            pltpu.async_copy(buf, out_hbm.at[pl.ds((base + c) * _CH, _CH), :], sem)

        def wait_write(c, buf, sem):
            pltpu.make_async_copy(
                buf, out_hbm.at[pl.ds((base + c) * _CH, _CH), :], sem).wait()

        n_pairs = chunks_pw // 2
        tail = chunks_pw % 2
        start_gather(0, buf0, sem_g0)

        def body(g, carry):
            c0 = 2 * g
            wait_gather(c0, buf0, sem_g0)
            start_write(c0, buf0, sem_w0)

            @pl.when(g >= 1)
            def _():
                wait_write(c0 - 1, buf1, sem_w1)

            start_gather(c0 + 1, buf1, sem_g1)
            wait_gather(c0 + 1, buf1, sem_g1)
            start_write(c0 + 1, buf1, sem_w1)
            wait_write(c0, buf0, sem_w0)
            cn = jnp.minimum(c0 + 2, chunks_pw - 1)
            start_gather(cn, buf0, sem_g0)
            return carry

        if n_pairs > 0:
            lax.fori_loop(0, n_pairs, body, 0)
        last = chunks_pw - 1
        wait_gather(last, buf0, sem_g0)
        if tail:
            start_write(last, buf0, sem_w0)
            wait_write(last, buf0, sem_w0)
        if n_pairs > 0:
            wait_write(last - tail, buf1, sem_w1)

    return gath


def _gather_ctx(table, idx):
    n = idx.shape[0]
    n_pad = -(-n // _ALIGN) * _ALIGN
    idx_pad = jnp.concatenate(
        [idx.astype(jnp.int32), jnp.zeros((n_pad - n,), jnp.int32)]
    )
    return _make_gather(n_pad, table.shape[1])(table, idx_pad)


def _accum(out_ref, i, loss_s, acc_s):
    l2 = lax.broadcasted_iota(jnp.int32, (1, 128), 1)
    upd = jnp.where(l2 == 0, loss_s, 0.0) + jnp.where(l2 == 1, acc_s, 0.0)

    @pl.when(i == 0)
    def _():
        out_ref[...] = jnp.zeros_like(out_ref)

    out_ref[...] += upd


def _softmax_body(T, N, vec_ref, ctx_ref, lab_ref, w1_ref, w2_ref, b2_ref, out_ref):
    i = pl.program_id(0)
    h = jnp.dot(vec_ref[...], w1_ref[...], preferred_element_type=jnp.float32)
    h = jnp.maximum(h + ctx_ref[...], 0.0)
    lt = lax.dot_general(w2_ref[...], h, (((1,), (1,)), ((), ())),
                         preferred_element_type=jnp.float32) + b2_ref[...]
    C = lt.shape[0]
    lab = lab_ref[...]
    sub = lax.broadcasted_iota(jnp.int32, lt.shape, 0)
    lse = jnp.log(jnp.sum(jnp.exp(lt), axis=0, keepdims=True))
    gold = jnp.sum(jnp.where(sub == lab, lt, 0.0), axis=0, keepdims=True)
    m = jnp.max(lt, axis=0, keepdims=True)
    amax = jnp.min(jnp.where(lt == m, sub, C), axis=0, keepdims=True)
    col = lax.broadcasted_iota(jnp.int32, (1, T), 1) + i * T
    valid = col < N
    loss_s = jnp.sum(jnp.where(valid, lse - gold, 0.0))
    acc_s = jnp.sum(jnp.where(valid & (amax == lab), 1.0, 0.0))
    _accum(out_ref, i, loss_s, acc_s)


def _topo_body(T, N, vec_ref, ctx_ref, lab_ref, w1_ref, w2_ref, b2_ref, out_ref):
    i = pl.program_id(0)
    h = jnp.dot(vec_ref[...], w1_ref[...], preferred_element_type=jnp.float32)
    h = jnp.maximum(h + ctx_ref[...], 0.0)
    z = lax.dot_general(w2_ref[...], h, (((1,), (1,)), ((), ())),
                        preferred_element_type=jnp.float32) + b2_ref[...]
    y = lab_ref[...]
    loss_row = jnp.maximum(z, 0.0) - z * y + jnp.log1p(jnp.exp(-jnp.abs(z)))
    corr = (z >= 0.0) == (y >= 0.5)
    col = lax.broadcasted_iota(jnp.int32, (1, T), 1) + i * T
    valid = col < N
    loss_s = jnp.sum(jnp.where(valid, loss_row, 0.0))
    acc_s = jnp.sum(jnp.where(valid & corr, 1.0, 0.0))
    _accum(out_ref, i, loss_s, acc_s)


def _fused_stream(body, vecs, ctx, lab2d, w1t, w2, b2, T=8192):
    N, D = vecs.shape
    H = w1t.shape[1]
    grid = -(-N // T)
    return pl.pallas_call(
        functools.partial(body, T, N),
        grid=(grid,),
        in_specs=[
            pl.BlockSpec((T, D), lambda i: (i, 0)),
            pl.BlockSpec((T, H), lambda i: (i, 0)),
            pl.BlockSpec((1, T), lambda i: (0, i)),
            pl.BlockSpec(w1t.shape, lambda i: (0, 0)),
            pl.BlockSpec(w2.shape, lambda i: (0, 0)),
            pl.BlockSpec(b2.shape, lambda i: (0, 0)),
        ],
        out_specs=pl.BlockSpec((1, 128), lambda i: (0, 0)),
        out_shape=jax.ShapeDtypeStruct((1, 128), jnp.float32),
    )(vecs, ctx, lab2d, w1t, w2, b2)


def kernel(src_graph_vecs, topo_vecs, atom_vecs, bond_vecs,
           W_t1, b_t1, W_t2, b_t2, W_a1, b_a1, W_a2, b_a2,
           W_b1, b_b1, W_b2, b_b2,
           topo_idx, atom_idx, bond_idx,
           topo_labels, atom_labels, bond_labels):
    B, L = src_graph_vecs.shape
    H = W_t1.shape[0]
    Dt = topo_vecs.shape[1]
    Da = atom_vecs.shape[1]
    Db = bond_vecs.shape[1]
    V = W_a2.shape[0]
    NBOND = W_b2.shape[0]
    NT, NA, NB = topo_idx.shape[0], atom_idx.shape[0], bond_idx.shape[0]

    pt, pa, pb = _project_tables(
        src_graph_vecs,
        W_t1[:, Dt:].T, W_a1[:, Da:].T, W_b1[:, Db:].T,
        b_t1[None, :], b_a1[None, :], b_b1[None, :],
    )

    ctx_a = _gather_ctx(pa, atom_idx)
    ctx_t = _gather_ctx(pt, topo_idx)
    ctx_b = _gather_ctx(pb, bond_idx)

    w2b = jnp.zeros((8, H), jnp.float32).at[:NBOND, :].set(W_b2)
    b2b = jnp.full((8, 1), -1e30, jnp.float32).at[:NBOND, 0].set(b_b2)

    a_out = _fused_stream(_softmax_body, atom_vecs, ctx_a,
                          atom_labels.astype(jnp.int32)[None, :],
                          W_a1[:, :Da].T, W_a2, b_a2[:, None])
    t_out = _fused_stream(_topo_body, topo_vecs, ctx_t,
                          topo_labels.astype(jnp.float32)[None, :],
                          W_t1[:, :Dt].T, W_t2, b_t2[:, None])
    b_out = _fused_stream(_softmax_body, bond_vecs, ctx_b,
                          bond_labels.astype(jnp.int32)[None, :],
                          W_b1[:, :Db].T, w2b, b2b)

    loss = (t_out[0, 0] + a_out[0, 0] + b_out[0, 0]) / B
    topo_acc = t_out[0, 1] / NT
    atom_acc = a_out[0, 1] / NA
    bond_acc = b_out[0, 1] / NB
    return (loss, atom_acc, topo_acc, bond_acc)
